# baseline (speedup 1.0000x reference)
import functools

import jax
import jax.numpy as jnp
from jax.experimental import pallas as pl
from jax.experimental.pallas import tpu as pltpu
from jax.experimental.pallas import tpu_sc as plsc


def _smooth_body(w3_ref, w5_ref, w7_ref, m3_ref, m5_ref, m7_ref, out_ref):
    r, d = w3_ref.shape
    for ref, m_ref, col in ((w3_ref, m3_ref, 0), (w5_ref, m5_ref, 64),
                            (w7_ref, m7_ref, 128)):
        out_ref[:, col:col + d] = jnp.dot(
            ref[...], m_ref[...], preferred_element_type=jnp.float32,
            precision=jax.lax.Precision.HIGHEST)
    out_ref[:, 3 * d:4 * d] = jnp.zeros((r, d), jnp.float32)


def _band_matrix(d, k):
    i = jnp.arange(d)
    band = (jnp.abs(i[:, None] - i[None, :]) <= (k - 1) // 2)
    return band.astype(jnp.float32) * (1.0 / k)


def _smooth_tables(w3, w5, w7):
    v, d = w3.shape
    blk = 5000
    grid = v // blk
    mats = [_band_matrix(d, k) for k in (3, 5, 7)]
    return pl.pallas_call(
        _smooth_body,
        grid=(grid,),
        in_specs=[pl.BlockSpec((blk, d), lambda i: (i, 0))] * 3
        + [pl.BlockSpec((d, d), lambda i: (0, 0))] * 3,
        out_specs=pl.BlockSpec((blk, 4 * d), lambda i: (i, 0)),
        out_shape=jax.ShapeDtypeStruct((v, 4 * d), jnp.float32),
    )(w3, w5, w7, *mats)


_NW = 32
_CHUNK = 400


def _sc_gather(table, idx):
    b = idx.shape[0]
    _, d = table.shape
    b_per_w = b // _NW
    n_chunks = b_per_w // _CHUNK
    mesh = plsc.VectorSubcoreMesh(core_axis_name="c", subcore_axis_name="s")

    @functools.partial(
        pl.kernel,
        out_type=jax.ShapeDtypeStruct((b, d), table.dtype),
        mesh=mesh,
        scratch_types=[
            pltpu.VMEM((_CHUNK,), jnp.int32),
            pltpu.VMEM((_CHUNK, d), jnp.float32),
            pltpu.SemaphoreType.DMA,
        ],
    )
    def gather_kernel(table_hbm, idx_hbm, out_hbm, idx_v, rows_v, sem):
        wid = jax.lax.axis_index("s") * 2 + jax.lax.axis_index("c")
        tile_base = wid * b_per_w

        @pl.loop(0, n_chunks)
        def _(c):
            base = tile_base + c * _CHUNK
            pltpu.sync_copy(idx_hbm.at[pl.ds(base, _CHUNK)], idx_v)
            pltpu.async_copy(table_hbm.at[idx_v], rows_v, sem).wait()
            pltpu.sync_copy(rows_v, out_hbm.at[pl.ds(base, _CHUNK)])

    return gather_kernel(table, idx)


def kernel(x, W3, W5, W7):
    bsz, seq = x.shape
    v, d = W3.shape
    wcat = _smooth_tables(W3, W5, W7)
    idx = x.reshape(-1).astype(jnp.int32)
    out = _sc_gather(wcat, idx)
    return out[:, :3 * d].reshape(bsz, seq, 3 * d)


# speedup vs baseline: 1.2251x; 1.2251x over previous
import functools

import jax
import jax.numpy as jnp
from jax.experimental import pallas as pl
from jax.experimental.pallas import tpu as pltpu
from jax.experimental.pallas import tpu_sc as plsc


def _smooth_body(w3_ref, w5_ref, w7_ref, m3_ref, m5_ref, m7_ref, out_ref):
    r, d = w3_ref.shape
    for ref, m_ref, col in ((w3_ref, m3_ref, 0), (w5_ref, m5_ref, 64),
                            (w7_ref, m7_ref, 128)):
        out_ref[:, col:col + d] = jnp.dot(
            ref[...], m_ref[...], preferred_element_type=jnp.float32,
            precision=jax.lax.Precision.DEFAULT)
    out_ref[:, 3 * d:4 * d] = jnp.zeros((r, d), jnp.float32)


def _band_matrix(d, k):
    i = jnp.arange(d)
    band = (jnp.abs(i[:, None] - i[None, :]) <= (k - 1) // 2)
    return band.astype(jnp.float32) * (1.0 / k)


def _smooth_tables(w3, w5, w7):
    v, d = w3.shape
    blk = 5000
    grid = v // blk
    mats = [_band_matrix(d, k) for k in (3, 5, 7)]
    return pl.pallas_call(
        _smooth_body,
        grid=(grid,),
        in_specs=[pl.BlockSpec((blk, d), lambda i: (i, 0))] * 3
        + [pl.BlockSpec((d, d), lambda i: (0, 0))] * 3,
        out_specs=pl.BlockSpec((blk, 4 * d), lambda i: (i, 0)),
        out_shape=jax.ShapeDtypeStruct((v, 4 * d), jnp.float32),
    )(w3, w5, w7, *mats)


_NW = 32
_CHUNK = 400


def _sc_gather(table, idx):
    b = idx.shape[0]
    _, d = table.shape
    b_per_w = b // _NW
    n_chunks = b_per_w // _CHUNK
    mesh = plsc.VectorSubcoreMesh(core_axis_name="c", subcore_axis_name="s")

    @functools.partial(
        pl.kernel,
        out_type=jax.ShapeDtypeStruct((b, d), table.dtype),
        mesh=mesh,
        scratch_types=[
            pltpu.VMEM((_CHUNK,), jnp.int32),
            pltpu.VMEM((_CHUNK, d), jnp.float32),
            pltpu.SemaphoreType.DMA,
        ],
    )
    def gather_kernel(table_hbm, idx_hbm, out_hbm, idx_v, rows_v, sem):
        wid = jax.lax.axis_index("s") * 2 + jax.lax.axis_index("c")
        tile_base = wid * b_per_w

        @pl.loop(0, n_chunks)
        def _(c):
            base = tile_base + c * _CHUNK
            pltpu.sync_copy(idx_hbm.at[pl.ds(base, _CHUNK)], idx_v)
            pltpu.async_copy(table_hbm.at[idx_v], rows_v, sem).wait()
            pltpu.sync_copy(rows_v, out_hbm.at[pl.ds(base, _CHUNK)])

    return gather_kernel(table, idx)


def kernel(x, W3, W5, W7):
    bsz, seq = x.shape
    v, d = W3.shape
    wcat = _smooth_tables(W3, W5, W7)
    idx = x.reshape(-1).astype(jnp.int32)
    out = _sc_gather(wcat, idx)
    return out[:, :3 * d].reshape(bsz, seq, 3 * d)


# speedup vs baseline: 1.2360x; 1.0089x over previous
import functools

import jax
import jax.numpy as jnp
from jax.experimental import pallas as pl
from jax.experimental.pallas import tpu as pltpu
from jax.experimental.pallas import tpu_sc as plsc


def _smooth_body(w3_ref, w5_ref, w7_ref, m3_ref, m5_ref, m7_ref, out_ref):
    r, d = w3_ref.shape
    for ref, m_ref, col in ((w3_ref, m3_ref, 0), (w5_ref, m5_ref, 64),
                            (w7_ref, m7_ref, 128)):
        out_ref[:, col:col + d] = jnp.dot(
            ref[...], m_ref[...], preferred_element_type=jnp.float32,
            precision=jax.lax.Precision.DEFAULT)
    out_ref[:, 3 * d:4 * d] = jnp.zeros((r, d), jnp.float32)


def _band_matrix(d, k):
    i = jnp.arange(d)
    band = (jnp.abs(i[:, None] - i[None, :]) <= (k - 1) // 2)
    return band.astype(jnp.float32) * (1.0 / k)


def _smooth_tables(w3, w5, w7):
    v, d = w3.shape
    blk = 5000
    grid = v // blk
    mats = [_band_matrix(d, k) for k in (3, 5, 7)]
    return pl.pallas_call(
        _smooth_body,
        grid=(grid,),
        in_specs=[pl.BlockSpec((blk, d), lambda i: (i, 0))] * 3
        + [pl.BlockSpec((d, d), lambda i: (0, 0))] * 3,
        out_specs=pl.BlockSpec((blk, 4 * d), lambda i: (i, 0)),
        out_shape=jax.ShapeDtypeStruct((v, 4 * d), jnp.float32),
    )(w3, w5, w7, *mats)


_NW = 32
_CHUNK = 200


def _sc_gather(table, idx):
    b = idx.shape[0]
    _, d = table.shape
    b_per_w = b // _NW
    n_chunks = b_per_w // _CHUNK
    n_pairs = n_chunks // 2
    mesh = plsc.VectorSubcoreMesh(core_axis_name="c", subcore_axis_name="s")

    @functools.partial(
        pl.kernel,
        out_type=jax.ShapeDtypeStruct((b, d), table.dtype),
        mesh=mesh,
        scratch_types=[
            pltpu.VMEM((_CHUNK,), jnp.int32),
            pltpu.VMEM((_CHUNK,), jnp.int32),
            pltpu.VMEM((_CHUNK, d), jnp.float32),
            pltpu.VMEM((_CHUNK, d), jnp.float32),
            pltpu.SemaphoreType.DMA,
            pltpu.SemaphoreType.DMA,
            pltpu.SemaphoreType.DMA,
            pltpu.SemaphoreType.DMA,
        ],
    )
    def gather_kernel(table_hbm, idx_hbm, out_hbm,
                      i0, i1, r0, r1, sg0, sg1, sw0, sw1):
        wid = jax.lax.axis_index("s") * 2 + jax.lax.axis_index("c")
        tile_base = wid * b_per_w

        @pl.loop(0, n_pairs)
        def _(p):
            base0 = tile_base + 2 * p * _CHUNK
            base1 = base0 + _CHUNK

            @pl.when(p > 0)
            def _():
                pltpu.make_async_copy(
                    r0, out_hbm.at[pl.ds(base0 - 2 * _CHUNK, _CHUNK)],
                    sw0).wait()
                pltpu.make_async_copy(
                    r1, out_hbm.at[pl.ds(base1 - 2 * _CHUNK, _CHUNK)],
                    sw1).wait()

            pltpu.sync_copy(idx_hbm.at[pl.ds(base0, _CHUNK)], i0)
            g0 = pltpu.async_copy(table_hbm.at[i0], r0, sg0)
            pltpu.sync_copy(idx_hbm.at[pl.ds(base1, _CHUNK)], i1)
            g1 = pltpu.async_copy(table_hbm.at[i1], r1, sg1)
            g0.wait()
            pltpu.async_copy(r0, out_hbm.at[pl.ds(base0, _CHUNK)], sw0)
            g1.wait()
            pltpu.async_copy(r1, out_hbm.at[pl.ds(base1, _CHUNK)], sw1)

        end0 = tile_base + (n_chunks - 2) * _CHUNK
        pltpu.make_async_copy(
            r0, out_hbm.at[pl.ds(end0, _CHUNK)], sw0).wait()
        pltpu.make_async_copy(
            r1, out_hbm.at[pl.ds(end0 + _CHUNK, _CHUNK)], sw1).wait()

    return gather_kernel(table, idx)


def kernel(x, W3, W5, W7):
    bsz, seq = x.shape
    v, d = W3.shape
    wcat = _smooth_tables(W3, W5, W7)
    idx = x.reshape(-1).astype(jnp.int32)
    out = _sc_gather(wcat, idx)
    return out[:, :3 * d].reshape(bsz, seq, 3 * d)
